# Initial kernel scaffold; baseline (speedup 1.0000x reference)
import jax
import jax.numpy as jnp
from jax.experimental import pallas as pl


def kernel(hidden_states, router_kernel, gate_up_proj, down_proj, shared_gate_kernel, shared_up_kernel, shared_down_kernel):
    raise NotImplementedError("write your pallas kernel here")


# speedup vs baseline: 4.4672x; 4.4672x over previous
import jax
import jax.numpy as jnp
from jax.experimental import pallas as pl

_SEQ, _HID, _FF, _E = 2048, 1024, 2048, 8
_CHUNK = _SEQ // _E
_LANES = 128


def _moe_body(x_ref, lg_ref, wg_ref, wu_ref, wsg_ref, wsu_ref, wd_ref,
              wsd_ref, out_ref, sc_ref):
    x32 = x_ref[...]

    logits = lg_ref[...]
    lane = jax.lax.broadcasted_iota(jnp.int32, logits.shape, 1)
    neg_inf = jnp.float32(-jnp.inf)
    l = jnp.where(lane < _E, logits, neg_inf)
    v1 = jnp.max(l, axis=1, keepdims=True)
    i1 = jnp.min(jnp.where(l == v1, lane, _LANES), axis=1, keepdims=True)
    l2 = jnp.where(lane == i1, neg_inf, l)
    v2 = jnp.max(l2, axis=1, keepdims=True)
    i2 = jnp.min(jnp.where(l2 == v2, lane, _LANES), axis=1, keepdims=True)
    sel = (lane == i1) | (lane == i2)
    sc_ref[...] = jnp.where(sel, jax.nn.sigmoid(l), 0.0)
    s1 = jax.nn.sigmoid(v1)
    s2 = jax.nn.sigmoid(v2)

    x16 = x32.astype(jnp.bfloat16)
    g = jnp.dot(x16, wg_ref[0], preferred_element_type=jnp.float32)
    u = jnp.dot(x16, wu_ref[0], preferred_element_type=jnp.float32)
    gs = jnp.dot(x16, wsg_ref[...], preferred_element_type=jnp.float32)
    us = jnp.dot(x16, wsu_ref[...], preferred_element_type=jnp.float32)

    a1 = s1 * g
    a2 = s2 * g
    h_routed = (s1 * u) * (a1 * jax.nn.sigmoid(a1)) \
             + (s2 * u) * (a2 * jax.nn.sigmoid(a2))
    h_shared = us * (gs * jax.nn.sigmoid(gs))

    out_ref[...] = (
        jnp.dot(h_routed.astype(jnp.bfloat16), wd_ref[0],
                preferred_element_type=jnp.float32)
        + jnp.dot(h_shared.astype(jnp.bfloat16), wsd_ref[...],
                  preferred_element_type=jnp.float32))


def kernel(hidden_states, router_kernel, gate_up_proj, down_proj,
           shared_gate_kernel, shared_up_kernel, shared_down_kernel):
    batch, seq, hid = hidden_states.shape
    flat = hidden_states.reshape(seq, hid)
    router_logits = flat @ router_kernel
    logits_pad = jnp.pad(router_logits, ((0, 0), (0, _LANES - _E)))
    gup16 = gate_up_proj.astype(jnp.bfloat16)
    wd16 = down_proj.astype(jnp.bfloat16)
    wsg16 = shared_gate_kernel.astype(jnp.bfloat16)
    wsu16 = shared_up_kernel.astype(jnp.bfloat16)
    wsd16 = shared_down_kernel.astype(jnp.bfloat16)

    out, scores = pl.pallas_call(
        _moe_body,
        grid=(_E,),
        in_specs=[
            pl.BlockSpec((_CHUNK, _HID), lambda c: (c, 0)),
            pl.BlockSpec((_CHUNK, _LANES), lambda c: (c, 0)),
            pl.BlockSpec((1, _HID, _FF), lambda c: (c, 0, 0)),
            pl.BlockSpec((1, _HID, _FF), lambda c: (c, 0, 1)),
            pl.BlockSpec((_HID, _FF), lambda c: (0, 0)),
            pl.BlockSpec((_HID, _FF), lambda c: (0, 0)),
            pl.BlockSpec((1, _FF, _HID), lambda c: (c, 0, 0)),
            pl.BlockSpec((_FF, _HID), lambda c: (0, 0)),
        ],
        out_specs=[
            pl.BlockSpec((_CHUNK, _HID), lambda c: (c, 0)),
            pl.BlockSpec((_CHUNK, _LANES), lambda c: (c, 0)),
        ],
        out_shape=[
            jax.ShapeDtypeStruct((seq, hid), jnp.float32),
            jax.ShapeDtypeStruct((seq, _LANES), jnp.float32),
        ],
    )(flat, logits_pad, gup16, gup16, wsg16, wsu16, wd16, wsd16)

    return out.reshape(batch, seq, hid), scores[:, :_E].T


# speedup vs baseline: 4.4690x; 1.0004x over previous
import jax
import jax.numpy as jnp
from jax.experimental import pallas as pl
from jax.experimental.pallas import tpu as pltpu

_SEQ, _HID, _FF, _E = 2048, 1024, 2048, 8
_CHUNK = _SEQ // _E
_LANES = 128


def _moe_body(x_ref, lg_ref, wg_ref, wu_ref, wsg_ref, wsu_ref, wd_ref,
              wsd_ref, out_ref, sc_ref):
    x32 = x_ref[...]

    logits = lg_ref[...]
    lane = jax.lax.broadcasted_iota(jnp.int32, logits.shape, 1)
    neg_inf = jnp.float32(-jnp.inf)
    l = jnp.where(lane < _E, logits, neg_inf)
    v1 = jnp.max(l, axis=1, keepdims=True)
    i1 = jnp.min(jnp.where(l == v1, lane, _LANES), axis=1, keepdims=True)
    l2 = jnp.where(lane == i1, neg_inf, l)
    v2 = jnp.max(l2, axis=1, keepdims=True)
    i2 = jnp.min(jnp.where(l2 == v2, lane, _LANES), axis=1, keepdims=True)
    sel = (lane == i1) | (lane == i2)
    sc_ref[...] = jnp.where(sel, jax.nn.sigmoid(l), 0.0)
    s1 = jax.nn.sigmoid(v1)
    s2 = jax.nn.sigmoid(v2)

    x16 = x32.astype(jnp.bfloat16)
    g = jnp.dot(x16, wg_ref[0], preferred_element_type=jnp.float32)
    u = jnp.dot(x16, wu_ref[0], preferred_element_type=jnp.float32)
    gs = jnp.dot(x16, wsg_ref[...], preferred_element_type=jnp.float32)
    us = jnp.dot(x16, wsu_ref[...], preferred_element_type=jnp.float32)

    a1 = s1 * g
    a2 = s2 * g
    h_routed = (s1 * u) * (a1 * jax.nn.sigmoid(a1)) \
             + (s2 * u) * (a2 * jax.nn.sigmoid(a2))
    h_shared = us * (gs * jax.nn.sigmoid(gs))

    out_ref[...] = (
        jnp.dot(h_routed.astype(jnp.bfloat16), wd_ref[0],
                preferred_element_type=jnp.float32)
        + jnp.dot(h_shared.astype(jnp.bfloat16), wsd_ref[...],
                  preferred_element_type=jnp.float32))


def kernel(hidden_states, router_kernel, gate_up_proj, down_proj,
           shared_gate_kernel, shared_up_kernel, shared_down_kernel):
    batch, seq, hid = hidden_states.shape
    flat = hidden_states.reshape(seq, hid)
    router_logits = flat @ router_kernel
    logits_pad = jnp.pad(router_logits, ((0, 0), (0, _LANES - _E)))
    gup16 = gate_up_proj.astype(jnp.bfloat16)
    wd16 = down_proj.astype(jnp.bfloat16)
    wsg16 = shared_gate_kernel.astype(jnp.bfloat16)
    wsu16 = shared_up_kernel.astype(jnp.bfloat16)
    wsd16 = shared_down_kernel.astype(jnp.bfloat16)

    out, scores = pl.pallas_call(
        _moe_body,
        grid=(_E,),
        in_specs=[
            pl.BlockSpec((_CHUNK, _HID), lambda c: (c, 0)),
            pl.BlockSpec((_CHUNK, _LANES), lambda c: (c, 0)),
            pl.BlockSpec((1, _HID, _FF), lambda c: (c, 0, 0)),
            pl.BlockSpec((1, _HID, _FF), lambda c: (c, 0, 1)),
            pl.BlockSpec((_HID, _FF), lambda c: (0, 0)),
            pl.BlockSpec((_HID, _FF), lambda c: (0, 0)),
            pl.BlockSpec((1, _FF, _HID), lambda c: (c, 0, 0)),
            pl.BlockSpec((_FF, _HID), lambda c: (0, 0)),
        ],
        out_specs=[
            pl.BlockSpec((_CHUNK, _HID), lambda c: (c, 0)),
            pl.BlockSpec((_CHUNK, _LANES), lambda c: (c, 0)),
        ],
        out_shape=[
            jax.ShapeDtypeStruct((seq, hid), jnp.float32),
            jax.ShapeDtypeStruct((seq, _LANES), jnp.float32),
        ],
        compiler_params=pltpu.CompilerParams(
            dimension_semantics=("parallel",)),
    )(flat, logits_pad, gup16, gup16, wsg16, wsu16, wd16, wsd16)

    return out.reshape(batch, seq, hid), scores[:, :_E].T


# speedup vs baseline: 6.6197x; 1.4812x over previous
import jax
import jax.numpy as jnp
from jax.experimental import pallas as pl
from jax.experimental.pallas import tpu as pltpu

_SEQ, _HID, _FF, _E = 2048, 1024, 2048, 8
_CHUNK = _SEQ // _E
_LANES = 128
_FT = 512
_FN = _FF // _FT


def _moe_body(x_ref, lg_ref, wg_ref, wu_ref, wsg_ref, wsu_ref, wd_ref,
              wsd_ref, out_ref, sc_ref, acc_ref):
    f = pl.program_id(0)
    c = pl.program_id(1)
    rows = pl.ds(c * _CHUNK, _CHUNK)

    logits = lg_ref[...]
    lane = jax.lax.broadcasted_iota(jnp.int32, logits.shape, 1)
    neg_inf = jnp.float32(-jnp.inf)
    l = jnp.where(lane < _E, logits, neg_inf)
    v1 = jnp.max(l, axis=1, keepdims=True)
    i1 = jnp.min(jnp.where(l == v1, lane, _LANES), axis=1, keepdims=True)
    l2 = jnp.where(lane == i1, neg_inf, l)
    v2 = jnp.max(l2, axis=1, keepdims=True)
    i2 = jnp.min(jnp.where(l2 == v2, lane, _LANES), axis=1, keepdims=True)
    s1 = jax.nn.sigmoid(v1)
    s2 = jax.nn.sigmoid(v2)

    @pl.when(f == 0)
    def _():
        sel = (lane == i1) | (lane == i2)
        sc_ref[...] = jnp.where(sel, jax.nn.sigmoid(l), 0.0)

    x16 = x_ref[...].astype(jnp.bfloat16)
    g = jnp.dot(x16, wg_ref[0].astype(jnp.bfloat16),
                preferred_element_type=jnp.float32)
    u = jnp.dot(x16, wu_ref[0].astype(jnp.bfloat16),
                preferred_element_type=jnp.float32)
    gs = jnp.dot(x16, wsg_ref[...].astype(jnp.bfloat16),
                 preferred_element_type=jnp.float32)
    us = jnp.dot(x16, wsu_ref[...].astype(jnp.bfloat16),
                 preferred_element_type=jnp.float32)

    a1 = s1 * g
    a2 = s2 * g
    h_routed = (s1 * u) * (a1 * jax.nn.sigmoid(a1)) \
             + (s2 * u) * (a2 * jax.nn.sigmoid(a2))
    h_shared = us * (gs * jax.nn.sigmoid(gs))

    partial = (
        jnp.dot(h_routed.astype(jnp.bfloat16), wd_ref[0].astype(jnp.bfloat16),
                preferred_element_type=jnp.float32)
        + jnp.dot(h_shared.astype(jnp.bfloat16),
                  wsd_ref[...].astype(jnp.bfloat16),
                  preferred_element_type=jnp.float32))

    @pl.when(f == 0)
    def _():
        acc_ref[rows, :] = partial

    @pl.when(f > 0)
    def _():
        acc_ref[rows, :] += partial

    @pl.when(f == _FN - 1)
    def _():
        out_ref[...] = acc_ref[rows, :]


def kernel(hidden_states, router_kernel, gate_up_proj, down_proj,
           shared_gate_kernel, shared_up_kernel, shared_down_kernel):
    batch, seq, hid = hidden_states.shape
    flat = hidden_states.reshape(seq, hid)
    router_logits = flat @ router_kernel
    logits_pad = jnp.pad(router_logits, ((0, 0), (0, _LANES - _E)))

    out, scores = pl.pallas_call(
        _moe_body,
        grid=(_FN, _E),
        in_specs=[
            pl.BlockSpec((_CHUNK, _HID), lambda f, c: (c, 0)),
            pl.BlockSpec((_CHUNK, _LANES), lambda f, c: (c, 0)),
            pl.BlockSpec((1, _HID, _FT), lambda f, c: (c, 0, f)),
            pl.BlockSpec((1, _HID, _FT), lambda f, c: (c, 0, f + _FN)),
            pl.BlockSpec((_HID, _FT), lambda f, c: (0, f)),
            pl.BlockSpec((_HID, _FT), lambda f, c: (0, f)),
            pl.BlockSpec((1, _FT, _HID), lambda f, c: (c, f, 0)),
            pl.BlockSpec((_FT, _HID), lambda f, c: (f, 0)),
        ],
        out_specs=[
            pl.BlockSpec((_CHUNK, _HID), lambda f, c: (c, 0)),
            pl.BlockSpec((_CHUNK, _LANES), lambda f, c: (c, 0)),
        ],
        out_shape=[
            jax.ShapeDtypeStruct((seq, hid), jnp.float32),
            jax.ShapeDtypeStruct((seq, _LANES), jnp.float32),
        ],
        scratch_shapes=[pltpu.VMEM((_SEQ, _HID), jnp.float32)],
        compiler_params=pltpu.CompilerParams(
            dimension_semantics=("arbitrary", "parallel")),
    )(flat, logits_pad, gate_up_proj, gate_up_proj,
      shared_gate_kernel, shared_up_kernel, down_proj, shared_down_kernel)

    return out.reshape(batch, seq, hid), scores[:, :_E].T


# speedup vs baseline: 7.0907x; 1.0711x over previous
import jax
import jax.numpy as jnp
from jax.experimental import pallas as pl
from jax.experimental.pallas import tpu as pltpu

_SEQ, _HID, _FF, _E = 2048, 1024, 2048, 8
_CHUNK = _SEQ // _E
_LANES = 128
_FT = 512
_FN = _FF // _FT


def _moe_body(x_ref, lg_ref, wg_ref, wu_ref, wsg_ref, wsu_ref, wd_ref,
              wsd_ref, out_ref, sc_ref, acc_ref):
    f = pl.program_id(0)
    c = pl.program_id(1)
    rows = pl.ds(c * _CHUNK, _CHUNK)

    logits = lg_ref[...]
    lane = jax.lax.broadcasted_iota(jnp.int32, logits.shape, 1)
    neg_inf = jnp.float32(-jnp.inf)
    l = jnp.where(lane < _E, logits, neg_inf)
    v1 = jnp.max(l, axis=1, keepdims=True)
    i1 = jnp.min(jnp.where(l == v1, lane, _LANES), axis=1, keepdims=True)
    l2 = jnp.where(lane == i1, neg_inf, l)
    v2 = jnp.max(l2, axis=1, keepdims=True)
    i2 = jnp.min(jnp.where(l2 == v2, lane, _LANES), axis=1, keepdims=True)
    s1 = jax.nn.sigmoid(v1)
    s2 = jax.nn.sigmoid(v2)

    sel = (lane == i1) | (lane == i2)
    sc_ref[...] = jnp.where(sel, jax.nn.sigmoid(l), 0.0)

    x16 = x_ref[...].astype(jnp.bfloat16)
    g = jnp.dot(x16, wg_ref[0].astype(jnp.bfloat16),
                preferred_element_type=jnp.float32)
    u = jnp.dot(x16, wu_ref[0].astype(jnp.bfloat16),
                preferred_element_type=jnp.float32)
    gs = jnp.dot(x16, wsg_ref[...].astype(jnp.bfloat16),
                 preferred_element_type=jnp.float32)
    us = jnp.dot(x16, wsu_ref[...].astype(jnp.bfloat16),
                 preferred_element_type=jnp.float32)

    a1 = s1 * g
    a2 = s2 * g
    h_routed = (s1 * u) * (a1 * jax.nn.sigmoid(a1)) \
             + (s2 * u) * (a2 * jax.nn.sigmoid(a2))
    h_shared = us * (gs * jax.nn.sigmoid(gs))

    partial = (
        jnp.dot(h_routed.astype(jnp.bfloat16), wd_ref[0].astype(jnp.bfloat16),
                preferred_element_type=jnp.float32)
        + jnp.dot(h_shared.astype(jnp.bfloat16),
                  wsd_ref[...].astype(jnp.bfloat16),
                  preferred_element_type=jnp.float32))

    @pl.when(f == 0)
    def _():
        acc_ref[rows, :] = partial

    @pl.when(f > 0)
    def _():
        acc_ref[rows, :] += partial

    @pl.when(f == _FN - 1)
    def _():
        out_ref[...] = acc_ref[rows, :]


def kernel(hidden_states, router_kernel, gate_up_proj, down_proj,
           shared_gate_kernel, shared_up_kernel, shared_down_kernel):
    batch, seq, hid = hidden_states.shape
    flat = hidden_states.reshape(seq, hid)
    router_logits = flat @ router_kernel
    logits_pad = jnp.pad(router_logits, ((0, 0), (0, _LANES - _E)))

    out, scores = pl.pallas_call(
        _moe_body,
        grid=(_FN, _E),
        in_specs=[
            pl.BlockSpec((_CHUNK, _HID), lambda f, c: (c, 0)),
            pl.BlockSpec((_CHUNK, _LANES), lambda f, c: (c, 0)),
            pl.BlockSpec((1, _HID, _FT), lambda f, c: (c, 0, f)),
            pl.BlockSpec((1, _HID, _FT), lambda f, c: (c, 0, f + _FN)),
            pl.BlockSpec((_HID, _FT), lambda f, c: (0, f)),
            pl.BlockSpec((_HID, _FT), lambda f, c: (0, f)),
            pl.BlockSpec((1, _FT, _HID), lambda f, c: (c, f, 0)),
            pl.BlockSpec((_FT, _HID), lambda f, c: (f, 0)),
        ],
        out_specs=[
            pl.BlockSpec((_CHUNK, _HID), lambda f, c: (c, 0)),
            pl.BlockSpec((_CHUNK, _LANES), lambda f, c: (c, 0)),
        ],
        out_shape=[
            jax.ShapeDtypeStruct((seq, hid), jnp.float32),
            jax.ShapeDtypeStruct((seq, _LANES), jnp.float32),
        ],
        scratch_shapes=[pltpu.VMEM((_SEQ, _HID), jnp.float32)],
        compiler_params=pltpu.CompilerParams(
            dimension_semantics=("arbitrary", "parallel")),
    )(flat, logits_pad, gate_up_proj, gate_up_proj,
      shared_gate_kernel, shared_up_kernel, down_proj, shared_down_kernel)

    return out.reshape(batch, seq, hid), scores[:, :_E].T


# speedup vs baseline: 7.1209x; 1.0043x over previous
import jax
import jax.numpy as jnp
from jax.experimental import pallas as pl
from jax.experimental.pallas import tpu as pltpu

_SEQ, _HID, _FF, _E = 2048, 1024, 2048, 8
_CHUNK = _SEQ // _E
_LANES = 128
_FT = 512
_FN = _FF // _FT


def _moe_body(x_ref, lg_ref, wg_ref, wu_ref, wsg_ref, wsu_ref, wd_ref,
              wsd_ref, out_ref, sc_ref, acc_ref):
    f = pl.program_id(0)
    c = pl.program_id(1)
    rows = pl.ds(c * _CHUNK, _CHUNK)

    logits = lg_ref[rows, :]
    lane = jax.lax.broadcasted_iota(jnp.int32, logits.shape, 1)
    neg_inf = jnp.float32(-jnp.inf)
    l = jnp.where(lane < _E, logits, neg_inf)
    v1 = jnp.max(l, axis=1, keepdims=True)
    i1 = jnp.min(jnp.where(l == v1, lane, _LANES), axis=1, keepdims=True)
    l2 = jnp.where(lane == i1, neg_inf, l)
    v2 = jnp.max(l2, axis=1, keepdims=True)
    i2 = jnp.min(jnp.where(l2 == v2, lane, _LANES), axis=1, keepdims=True)
    s1 = jax.nn.sigmoid(v1)
    s2 = jax.nn.sigmoid(v2)

    sel = (lane == i1) | (lane == i2)
    sc_ref[...] = jnp.where(sel, jax.nn.sigmoid(l), 0.0)

    x16 = x_ref[rows, :].astype(jnp.bfloat16)
    g = jnp.dot(x16, wg_ref[0].astype(jnp.bfloat16),
                preferred_element_type=jnp.float32)
    u = jnp.dot(x16, wu_ref[0].astype(jnp.bfloat16),
                preferred_element_type=jnp.float32)
    gs = jnp.dot(x16, wsg_ref[...].astype(jnp.bfloat16),
                 preferred_element_type=jnp.float32)
    us = jnp.dot(x16, wsu_ref[...].astype(jnp.bfloat16),
                 preferred_element_type=jnp.float32)

    a1 = s1 * g
    a2 = s2 * g
    h_routed = (s1 * u) * (a1 * jax.nn.sigmoid(a1)) \
             + (s2 * u) * (a2 * jax.nn.sigmoid(a2))
    h_shared = us * (gs * jax.nn.sigmoid(gs))

    partial = (
        jnp.dot(h_routed.astype(jnp.bfloat16), wd_ref[0].astype(jnp.bfloat16),
                preferred_element_type=jnp.float32)
        + jnp.dot(h_shared.astype(jnp.bfloat16),
                  wsd_ref[...].astype(jnp.bfloat16),
                  preferred_element_type=jnp.float32))

    @pl.when(f == 0)
    def _():
        acc_ref[rows, :] = partial

    @pl.when(f > 0)
    def _():
        acc_ref[rows, :] += partial

    @pl.when(f == _FN - 1)
    def _():
        out_ref[...] = acc_ref[rows, :]


def kernel(hidden_states, router_kernel, gate_up_proj, down_proj,
           shared_gate_kernel, shared_up_kernel, shared_down_kernel):
    batch, seq, hid = hidden_states.shape
    flat = hidden_states.reshape(seq, hid)
    router_logits = flat @ router_kernel
    logits_pad = jnp.pad(router_logits, ((0, 0), (0, _LANES - _E)))

    out, scores = pl.pallas_call(
        _moe_body,
        grid=(_FN, _E),
        in_specs=[
            pl.BlockSpec((_SEQ, _HID), lambda f, c: (0, 0)),
            pl.BlockSpec((_SEQ, _LANES), lambda f, c: (0, 0)),
            pl.BlockSpec((1, _HID, _FT), lambda f, c: (c, 0, f)),
            pl.BlockSpec((1, _HID, _FT), lambda f, c: (c, 0, f + _FN)),
            pl.BlockSpec((_HID, _FT), lambda f, c: (0, f)),
            pl.BlockSpec((_HID, _FT), lambda f, c: (0, f)),
            pl.BlockSpec((1, _FT, _HID), lambda f, c: (c, f, 0)),
            pl.BlockSpec((_FT, _HID), lambda f, c: (f, 0)),
        ],
        out_specs=[
            pl.BlockSpec((_CHUNK, _HID), lambda f, c: (c, 0)),
            pl.BlockSpec((_CHUNK, _LANES), lambda f, c: (c, 0)),
        ],
        out_shape=[
            jax.ShapeDtypeStruct((seq, hid), jnp.float32),
            jax.ShapeDtypeStruct((seq, _LANES), jnp.float32),
        ],
        scratch_shapes=[pltpu.VMEM((_SEQ, _HID), jnp.float32)],
        compiler_params=pltpu.CompilerParams(
            dimension_semantics=("arbitrary", "parallel")),
    )(flat, logits_pad, gate_up_proj, gate_up_proj,
      shared_gate_kernel, shared_up_kernel, down_proj, shared_down_kernel)

    return out.reshape(batch, seq, hid), scores[:, :_E].T


# speedup vs baseline: 7.5203x; 1.0561x over previous
import jax
import jax.numpy as jnp
from jax.experimental import pallas as pl
from jax.experimental.pallas import tpu as pltpu

_SEQ, _HID, _FF, _E = 2048, 1024, 2048, 8
_CHUNK = _SEQ // _E
_LANES = 128
_FT = 512
_FN = _FF // _FT


def _moe_body(x_ref, lg_ref, wg_ref, wu_ref, wsg_ref, wsu_ref, wd_ref,
              wsd_ref, out_ref, sc_ref):
    c = pl.program_id(0)
    f = pl.program_id(1)
    rows = pl.ds(c * _CHUNK, _CHUNK)
    cols = pl.ds(f * _FT, _FT)

    logits = lg_ref[rows, :]
    lane = jax.lax.broadcasted_iota(jnp.int32, logits.shape, 1)
    neg_inf = jnp.float32(-jnp.inf)
    l = jnp.where(lane < _E, logits, neg_inf)
    v1 = jnp.max(l, axis=1, keepdims=True)
    i1 = jnp.min(jnp.where(l == v1, lane, _LANES), axis=1, keepdims=True)
    l2 = jnp.where(lane == i1, neg_inf, l)
    v2 = jnp.max(l2, axis=1, keepdims=True)
    i2 = jnp.min(jnp.where(l2 == v2, lane, _LANES), axis=1, keepdims=True)
    s1 = jax.nn.sigmoid(v1)
    s2 = jax.nn.sigmoid(v2)

    sel = (lane == i1) | (lane == i2)
    sc_ref[...] = jnp.where(sel, jax.nn.sigmoid(l), 0.0)

    x16 = x_ref[rows, :].astype(jnp.bfloat16)
    g = jnp.dot(x16, wg_ref[0].astype(jnp.bfloat16),
                preferred_element_type=jnp.float32)
    u = jnp.dot(x16, wu_ref[0].astype(jnp.bfloat16),
                preferred_element_type=jnp.float32)
    gs = jnp.dot(x16, wsg_ref[:, cols].astype(jnp.bfloat16),
                 preferred_element_type=jnp.float32)
    us = jnp.dot(x16, wsu_ref[:, cols].astype(jnp.bfloat16),
                 preferred_element_type=jnp.float32)

    a1 = s1 * g
    a2 = s2 * g
    h_routed = (s1 * u) * (a1 * jax.nn.sigmoid(a1)) \
             + (s2 * u) * (a2 * jax.nn.sigmoid(a2))
    h_shared = us * (gs * jax.nn.sigmoid(gs))

    partial = (
        jnp.dot(h_routed.astype(jnp.bfloat16), wd_ref[0].astype(jnp.bfloat16),
                preferred_element_type=jnp.float32)
        + jnp.dot(h_shared.astype(jnp.bfloat16),
                  wsd_ref[cols, :].astype(jnp.bfloat16),
                  preferred_element_type=jnp.float32))

    @pl.when(f == 0)
    def _():
        out_ref[...] = partial

    @pl.when(f > 0)
    def _():
        out_ref[...] += partial


def kernel(hidden_states, router_kernel, gate_up_proj, down_proj,
           shared_gate_kernel, shared_up_kernel, shared_down_kernel):
    batch, seq, hid = hidden_states.shape
    flat = hidden_states.reshape(seq, hid)
    router_logits = flat @ router_kernel
    logits_pad = jnp.pad(router_logits, ((0, 0), (0, _LANES - _E)))

    out, scores = pl.pallas_call(
        _moe_body,
        grid=(_E, _FN),
        in_specs=[
            pl.BlockSpec((_SEQ, _HID), lambda c, f: (0, 0)),
            pl.BlockSpec((_SEQ, _LANES), lambda c, f: (0, 0)),
            pl.BlockSpec((1, _HID, _FT), lambda c, f: (c, 0, f)),
            pl.BlockSpec((1, _HID, _FT), lambda c, f: (c, 0, f + _FN)),
            pl.BlockSpec((_HID, _FF), lambda c, f: (0, 0)),
            pl.BlockSpec((_HID, _FF), lambda c, f: (0, 0)),
            pl.BlockSpec((1, _FT, _HID), lambda c, f: (c, f, 0)),
            pl.BlockSpec((_FF, _HID), lambda c, f: (0, 0)),
        ],
        out_specs=[
            pl.BlockSpec((_CHUNK, _HID), lambda c, f: (c, 0)),
            pl.BlockSpec((_CHUNK, _LANES), lambda c, f: (c, 0)),
        ],
        out_shape=[
            jax.ShapeDtypeStruct((seq, hid), jnp.float32),
            jax.ShapeDtypeStruct((seq, _LANES), jnp.float32),
        ],
        compiler_params=pltpu.CompilerParams(
            dimension_semantics=("parallel", "arbitrary")),
    )(flat, logits_pad, gate_up_proj, gate_up_proj,
      shared_gate_kernel, shared_up_kernel, down_proj, shared_down_kernel)

    return out.reshape(batch, seq, hid), scores[:, :_E].T
